# baseline (speedup 1.0000x reference)
import functools

import jax
import jax.numpy as jnp
from jax.experimental import pallas as pl
from jax.experimental.pallas import tpu as pltpu

X_POS = 5
CTX = 77
D = 512
BB = 128


def _gather_body(ids_ref, table_ref, out_ref):
    out_ref[...] = table_ref[...]


def _compose_body(emb_ref, s_ref, tok_ref, out_ref, tok_out_ref):
    bb = s_ref.shape[0]
    emb = emb_ref[...]
    full = jnp.broadcast_to(emb[None, :, :], (bb, CTX, D))
    row = jax.lax.broadcasted_iota(jnp.int32, (bb, CTX, D), 1)
    s = s_ref[...][:, None, :]
    out_ref[...] = jnp.where(row == X_POS, s, full)
    tok_out_ref[...] = jnp.broadcast_to(tok_ref[...], (bb, CTX))


@jax.jit
def kernel(s_star, tokenized_composed, table):
    b = s_star.shape[0]
    ids = tokenized_composed.reshape(CTX)

    emb = pl.pallas_call(
        _gather_body,
        grid_spec=pltpu.PrefetchScalarGridSpec(
            num_scalar_prefetch=1,
            grid=(CTX,),
            in_specs=[pl.BlockSpec((1, 1, D), lambda i, ids: (ids[i], 0, 0))],
            out_specs=pl.BlockSpec((1, 1, D), lambda i, ids: (i, 0, 0)),
        ),
        out_shape=jax.ShapeDtypeStruct((CTX, 1, D), table.dtype),
    )(ids, table.reshape(table.shape[0], 1, D)).reshape(CTX, D)

    nb = b // BB
    prompts, tokenized = pl.pallas_call(
        _compose_body,
        grid=(nb,),
        in_specs=[
            pl.BlockSpec((CTX, D), lambda i: (0, 0)),
            pl.BlockSpec((BB, D), lambda i: (i, 0)),
            pl.BlockSpec((1, CTX), lambda i: (0, 0)),
        ],
        out_specs=[
            pl.BlockSpec((BB, CTX, D), lambda i: (i, 0, 0)),
            pl.BlockSpec((BB, CTX), lambda i: (i, 0)),
        ],
        out_shape=[
            jax.ShapeDtypeStruct((b, CTX, D), jnp.float32),
            jax.ShapeDtypeStruct((b, CTX), jnp.int32),
        ],
    )(emb, s_star.astype(jnp.float32), tokenized_composed)

    return (prompts, tokenized)


# speedup vs baseline: 1.1768x; 1.1768x over previous
import jax
import jax.numpy as jnp
from jax.experimental import pallas as pl
from jax.experimental.pallas import tpu as pltpu

X_POS = 5
CTX = 77
D = 512
BB = 128


def _body(ids_ref, table_ref, s_ref, tok_ref, out_ref, tok_out_ref,
          emb_scr, sem):
    i = pl.program_id(0)

    @pl.when(i == 0)
    def _gather():
        def start(k, carry):
            pltpu.make_async_copy(
                table_ref.at[pl.ds(ids_ref[k], 1)],
                emb_scr.at[pl.ds(k, 1)],
                sem,
            ).start()
            return carry

        jax.lax.fori_loop(0, CTX, start, 0)

        def wait(k, carry):
            pltpu.make_async_copy(
                table_ref.at[pl.ds(ids_ref[k], 1)],
                emb_scr.at[pl.ds(k, 1)],
                sem,
            ).wait()
            return carry

        jax.lax.fori_loop(0, CTX, wait, 0)

    emb = emb_scr[...]
    full = jnp.broadcast_to(emb[None, :, :], (BB, CTX, D))
    row = jax.lax.broadcasted_iota(jnp.int32, (BB, CTX, D), 1)
    s = s_ref[...][:, None, :]
    out_ref[...] = jnp.where(row == X_POS, s, full)
    tok_out_ref[...] = jnp.broadcast_to(tok_ref[...], (BB, CTX))


@jax.jit
def kernel(s_star, tokenized_composed, table):
    b = s_star.shape[0]
    ids = tokenized_composed.reshape(CTX)
    nb = b // BB

    prompts, tokenized = pl.pallas_call(
        _body,
        grid_spec=pltpu.PrefetchScalarGridSpec(
            num_scalar_prefetch=1,
            grid=(nb,),
            in_specs=[
                pl.BlockSpec(memory_space=pltpu.MemorySpace.HBM),
                pl.BlockSpec((BB, D), lambda i, ids: (i, 0)),
                pl.BlockSpec((1, CTX), lambda i, ids: (0, 0)),
            ],
            out_specs=[
                pl.BlockSpec((BB, CTX, D), lambda i, ids: (i, 0, 0)),
                pl.BlockSpec((BB, CTX), lambda i, ids: (i, 0)),
            ],
            scratch_shapes=[
                pltpu.VMEM((CTX, D), jnp.float32),
                pltpu.SemaphoreType.DMA,
            ],
        ),
        out_shape=[
            jax.ShapeDtypeStruct((b, CTX, D), jnp.float32),
            jax.ShapeDtypeStruct((b, CTX), jnp.int32),
        ],
    )(ids, table, s_star.astype(jnp.float32), tokenized_composed)

    return (prompts, tokenized)
